# baseline (speedup 1.0000x reference)
import functools

import jax
import jax.numpy as jnp
from jax import lax
from jax.experimental import pallas as pl
from jax.experimental.pallas import tpu as pltpu
from jax.experimental.pallas import tpu_sc as plsc

BATCH = 16384
FIELDS = 26
EMBED_DIM = 128

_TOTAL = BATCH * FIELDS
_BLK = 104
_NBLK = _TOTAL // _BLK
_NW = 32
_BPW = _NBLK // _NW
_NBUF = 8
_NGRP = _BPW // _NBUF


def _gather_kernel(table_hbm, idx_hbm, out_hbm, idx_v, *bufs_and_sems):
    rows = bufs_and_sems[:_NBUF]
    gsem = bufs_and_sems[_NBUF:2 * _NBUF]
    wsem = bufs_and_sems[2 * _NBUF:3 * _NBUF]

    nc = 2
    wid = lax.axis_index("s") * nc + lax.axis_index("c")
    base = wid * _BPW
    pltpu.sync_copy(idx_hbm.at[pl.ds(base, _BPW)], idx_v)

    def start_gather(j, b):
        pltpu.make_async_copy(table_hbm.at[idx_v.at[j]], rows[b], gsem[b]).start()

    def start_writeback(j, b):
        pltpu.make_async_copy(
            rows[b], out_hbm.at[pl.ds((base + j) * _BLK, _BLK)], wsem[b]
        ).start()

    def wait_gather(b):
        pltpu.make_async_copy(table_hbm.at[idx_v.at[0]], rows[b], gsem[b]).wait()

    def wait_writeback(j, b):
        pltpu.make_async_copy(
            rows[b], out_hbm.at[pl.ds((base + j) * _BLK, _BLK)], wsem[b]
        ).wait()

    for b in range(_NBUF):
        start_gather(b, b)

    def grp(g, carry):
        j0 = g * _NBUF
        for b in range(_NBUF):
            wait_gather(b)
            start_writeback(j0 + b, b)
        for b in range(_NBUF):
            wait_writeback(j0 + b, b)
            start_gather(j0 + _NBUF + b, b)
        return carry

    lax.fori_loop(0, _NGRP - 1, grp, 0)

    j0 = (_NGRP - 1) * _NBUF
    for b in range(_NBUF):
        wait_gather(b)
        start_writeback(j0 + b, b)
    for b in range(_NBUF):
        wait_writeback(j0 + b, b)


@jax.jit
def _run(idx2d, table):
    mesh = plsc.VectorSubcoreMesh(core_axis_name="c", subcore_axis_name="s")
    scratch = (
        [pltpu.VMEM((_BPW, _BLK), jnp.int32)]
        + [pltpu.VMEM((_BLK, EMBED_DIM), jnp.float32) for _ in range(_NBUF)]
        + [pltpu.SemaphoreType.DMA for _ in range(2 * _NBUF)]
    )
    kfn = functools.partial(
        pl.kernel,
        mesh=mesh,
        out_type=jax.ShapeDtypeStruct((_TOTAL, EMBED_DIM), jnp.float32),
        scratch_types=scratch,
    )(_gather_kernel)
    return kfn(table, idx2d)


def kernel(inputs, table):
    idx2d = inputs.T.reshape(_NBLK, _BLK).astype(jnp.int32)
    out = _run(idx2d, table)
    return out.reshape(FIELDS, BATCH, EMBED_DIM).transpose(1, 0, 2)


# speedup vs baseline: 1.0104x; 1.0104x over previous
import functools

import jax
import jax.numpy as jnp
from jax import lax
from jax.experimental import pallas as pl
from jax.experimental.pallas import tpu as pltpu
from jax.experimental.pallas import tpu_sc as plsc

BATCH = 16384
FIELDS = 26
EMBED_DIM = 128

_TOTAL = BATCH * FIELDS
_BLK = 128
_NBLK = _TOTAL // _BLK
_NW = 32
_BPW = _NBLK // _NW
_NBUF = 4
_NGRP = _BPW // _NBUF


def _gather_kernel(table_hbm, idx_hbm, out_hbm, idx_v, *bufs_and_sems):
    rows = bufs_and_sems[:_NBUF]
    gsem = bufs_and_sems[_NBUF:2 * _NBUF]
    wsem = bufs_and_sems[2 * _NBUF:3 * _NBUF]

    nc = 2
    wid = lax.axis_index("s") * nc + lax.axis_index("c")
    base = wid * _BPW
    pltpu.sync_copy(idx_hbm.at[pl.ds(base, _BPW)], idx_v)

    def start_gather(j, b):
        pltpu.make_async_copy(table_hbm.at[idx_v.at[j]], rows[b], gsem[b]).start()

    def start_writeback(j, b):
        pltpu.make_async_copy(
            rows[b], out_hbm.at[pl.ds((base + j) * _BLK, _BLK)], wsem[b]
        ).start()

    def wait_gather(b):
        pltpu.make_async_copy(table_hbm.at[idx_v.at[0]], rows[b], gsem[b]).wait()

    def wait_writeback(j, b):
        pltpu.make_async_copy(
            rows[b], out_hbm.at[pl.ds((base + j) * _BLK, _BLK)], wsem[b]
        ).wait()

    for b in range(_NBUF):
        start_gather(b, b)

    def grp(g, carry):
        j0 = g * _NBUF
        for b in range(_NBUF):
            wait_gather(b)
            start_writeback(j0 + b, b)
        for b in range(_NBUF):
            wait_writeback(j0 + b, b)
            start_gather(j0 + _NBUF + b, b)
        return carry

    lax.fori_loop(0, _NGRP - 1, grp, 0)

    j0 = (_NGRP - 1) * _NBUF
    for b in range(_NBUF):
        wait_gather(b)
        start_writeback(j0 + b, b)
    for b in range(_NBUF):
        wait_writeback(j0 + b, b)


@jax.jit
def _run(idx2d, table):
    mesh = plsc.VectorSubcoreMesh(core_axis_name="c", subcore_axis_name="s")
    scratch = (
        [pltpu.VMEM((_BPW, _BLK), jnp.int32)]
        + [pltpu.VMEM((_BLK, EMBED_DIM), jnp.float32) for _ in range(_NBUF)]
        + [pltpu.SemaphoreType.DMA for _ in range(2 * _NBUF)]
    )
    kfn = functools.partial(
        pl.kernel,
        mesh=mesh,
        out_type=jax.ShapeDtypeStruct((_TOTAL, EMBED_DIM), jnp.float32),
        scratch_types=scratch,
    )(_gather_kernel)
    return kfn(table, idx2d)


def kernel(inputs, table):
    idx2d = inputs.T.reshape(_NBLK, _BLK).astype(jnp.int32)
    out = _run(idx2d, table)
    return out.reshape(FIELDS, BATCH, EMBED_DIM).transpose(1, 0, 2)
